# baseline (speedup 1.0000x reference)
import jax
import jax.numpy as jnp
from jax import lax
from jax.experimental import pallas as pl
from jax.experimental.pallas import tpu as pltpu
from jax.experimental.pallas import tpu_sc as plsc

NUM_INTENTS = 100000
EMBED_DIM = 64
BATCH = 16384
SEQ_LEN = 200

GLEN = 100
GPB = SEQ_LEN // GLEN
NW = 32
B_PER_W = BATCH // NW
CB = 4
NG = CB * GPB
GROUPS = B_PER_W // CB
NIDXROWS = BATCH * GPB


def _gather_body(table_hbm, idx_hbm, out_hbm, idx_v, rows_v,
                 gsem0, gsem1, ssem0, ssem1):
    wid = lax.axis_index("s") * 2 + lax.axis_index("c")
    base_b = wid * B_PER_W
    gsems = (gsem0, gsem1)
    ssems = (ssem0, ssem1)

    def load_idx(g, b):
        u0 = (base_b + g * CB) * GPB
        pltpu.sync_copy(idx_hbm.at[pl.ds(u0, NG)], idx_v.at[b])

    def fire_gathers(b):
        for j in range(NG):
            dst = rows_v.at[b].at[j // GPB].at[pl.ds((j % GPB) * GLEN, GLEN)]
            pltpu.async_copy(table_hbm.at[idx_v.at[b].at[j]], dst, gsems[b])

    def wait_gathers(b):
        for j in range(NG):
            dst = rows_v.at[b].at[j // GPB].at[pl.ds((j % GPB) * GLEN, GLEN)]
            pltpu.make_async_copy(table_hbm.at[pl.ds(0, GLEN)], dst,
                                  gsems[b]).wait()

    def store(g, b):
        pltpu.async_copy(rows_v.at[b], out_hbm.at[pl.ds(base_b + g * CB, CB)],
                         ssems[b])

    def wait_store(b):
        pltpu.make_async_copy(rows_v.at[b], out_hbm.at[pl.ds(0, CB)],
                              ssems[b]).wait()

    load_idx(0, 0)
    fire_gathers(0)
    load_idx(1, 1)
    fire_gathers(1)
    wait_gathers(0)
    store(0, 0)

    def loop_body(t, carry):
        for b in range(2):
            g = 2 * t + 2 + b
            b2 = 1 - b
            wait_store(b)
            load_idx(g, b)
            fire_gathers(b)
            wait_gathers(b2)
            store(g - 1, b2)
        return carry

    lax.fori_loop(0, (GROUPS - 2) // 2, loop_body, 0)

    last_b = (GROUPS - 1) % 2
    wait_gathers(last_b)
    store(GROUPS - 1, last_b)
    wait_store(1 - last_b)
    wait_store(last_b)


@jax.jit
def _gather(table, idx2d):
    mesh = plsc.VectorSubcoreMesh(core_axis_name="c", subcore_axis_name="s")
    return pl.kernel(
        _gather_body,
        mesh=mesh,
        out_type=jax.ShapeDtypeStruct((BATCH, SEQ_LEN, EMBED_DIM),
                                      jnp.float32),
        scratch_types=[
            pltpu.VMEM((2, NG, GLEN), jnp.int32),
            pltpu.VMEM((2, CB, SEQ_LEN, EMBED_DIM), jnp.float32),
            pltpu.SemaphoreType.DMA,
            pltpu.SemaphoreType.DMA,
            pltpu.SemaphoreType.DMA,
            pltpu.SemaphoreType.DMA,
        ],
        compiler_params=pltpu.CompilerParams(use_tc_tiling_on_sc=False),
    )(table, idx2d)


def kernel(intent_ids, table):
    idx2d = intent_ids.reshape(NIDXROWS, GLEN)
    return _gather(table, idx2d)
